# Initial kernel scaffold; baseline (speedup 1.0000x reference)
import jax
import jax.numpy as jnp
from jax.experimental import pallas as pl


def kernel(x, edge_index, batch, edge_attr, W1, att_src1, att_dst1, We1, att_e1, b1, W2, att_src2, att_dst2, We2, att_e2, b2):
    raise NotImplementedError("write your pallas kernel here")


# speedup vs baseline: 1.1211x; 1.1211x over previous
import functools

import jax
import jax.numpy as jnp
from jax.experimental import pallas as pl


def _matmul_kernel(x_ref, w_ref, o_ref):
    o_ref[...] = jnp.dot(x_ref[...], w_ref[...], preferred_element_type=jnp.float32)


def _matmul(x, w, block_rows=1000):
    n, k = x.shape
    m = w.shape[1]
    grid = (n // block_rows,)
    return pl.pallas_call(
        _matmul_kernel,
        grid=grid,
        in_specs=[
            pl.BlockSpec((block_rows, k), lambda i: (i, 0)),
            pl.BlockSpec((k, m), lambda i: (0, 0)),
        ],
        out_specs=pl.BlockSpec((block_rows, m), lambda i: (i, 0)),
        out_shape=jax.ShapeDtypeStruct((n, m), jnp.float32),
    )(x, w)


def _gat_layer(x, src, dst, edge_attr, W, a_src, a_dst, We, a_edge, b, H, C):
    n = x.shape[0]
    xp = _matmul(x, W)
    xph = xp.reshape(n, H, C)
    alpha_src = jnp.einsum("nhc,hc->nh", xph, a_src[0])
    alpha_dst = jnp.einsum("nhc,hc->nh", xph, a_dst[0])
    We_eff = jnp.einsum("dhc,hc->dh", We.reshape(-1, H, C), a_edge[0])
    alpha_e = edge_attr @ We_eff
    alpha = alpha_src[src] + alpha_dst[dst] + alpha_e
    alpha = jnp.maximum(alpha, 0.2 * alpha)
    aexp = jnp.exp(alpha)
    denom = jnp.zeros((n, H), jnp.float32).at[dst].add(aexp)
    msg = xph[src] * aexp[:, :, None]
    acc = jnp.zeros((n, H, C), jnp.float32).at[dst].add(msg)
    out = acc / (denom[:, :, None] + 1e-16)
    return out.reshape(n, H * C) + b


def kernel(x, edge_index, batch, edge_attr,
           W1, att_src1, att_dst1, We1, att_e1, b1,
           W2, att_src2, att_dst2, We2, att_e2, b2):
    src = edge_index[0]
    dst = edge_index[1]
    H1, C1 = att_src1.shape[1], att_src1.shape[2]
    H2, C2 = att_src2.shape[1], att_src2.shape[2]
    G = 64
    h = _gat_layer(x, src, dst, edge_attr, W1, att_src1, att_dst1, We1, att_e1, b1, H1, C1)
    h = jax.nn.elu(h)
    h = _gat_layer(h, src, dst, edge_attr, W2, att_src2, att_dst2, We2, att_e2, b2, H2, C2)
    sums = jnp.zeros((G, h.shape[1]), jnp.float32).at[batch].add(h)
    cnt = jnp.zeros((G,), jnp.float32).at[batch].add(1.0)
    return sums / jnp.clip(cnt, 1.0, None)[:, None]


# speedup vs baseline: 36.2487x; 32.3325x over previous
import functools

import jax
import jax.numpy as jnp
from jax import lax
from jax.experimental import pallas as pl
from jax.experimental.pallas import tpu as pltpu
from jax.experimental.pallas import tpu_sc as plsc

NC = 2
NS = 16
NW = NC * NS


def _sc_edge_aggregate(H, C, N, E):
    HC = H * C
    W_ACC = HC + 16
    K = 80
    EW = E // NW
    NCH = EW // K
    assert EW * NW == E and NCH * K == EW
    CPR = 80
    NCPT = N // CPR
    TMAX = -(-NCPT // NS)
    assert NCPT * CPR == N
    mesh = plsc.VectorSubcoreMesh(core_axis_name="c", subcore_axis_name="s",
                                  num_cores=NC, num_subcores=NS)

    @functools.partial(
        pl.kernel,
        out_type=jax.ShapeDtypeStruct((NC, N, W_ACC), jnp.float32),
        mesh=mesh,
        compiler_params=pltpu.CompilerParams(use_tc_tiling_on_sc=False),
        scratch_types=[
            pltpu.VMEM((K,), jnp.int32),
            pltpu.VMEM((K,), jnp.int32),
            pltpu.VMEM((K * H + 16,), jnp.float32),
            pltpu.VMEM((K, W_ACC), jnp.float32),
            pltpu.VMEM((K, 16), jnp.float32),
            pltpu.VMEM((K, W_ACC), jnp.float32),
            pltpu.VMEM((CPR, W_ACC), jnp.float32),
            pltpu.VMEM_SHARED((N, W_ACC), jnp.float32),
            pltpu.SemaphoreType.DMA,
            pltpu.SemaphoreType.DMA,
        ],
    )
    def body(t_hbm, ad_hbm, ae_hbm, src_hbm, dst_hbm, out_hbm,
             src_v, dst_v, ae_v, t_v, ad_v, msg_v, cp_v, acc_sh,
             sem_a, sem_b):
        c = lax.axis_index("c")
        s = lax.axis_index("s")
        iota = lax.iota(jnp.int32, 16)
        zeros16 = jnp.zeros((16,), jnp.float32)
        log2c = C.bit_length() - 1
        hol = [(j * 16 + iota) >> log2c for j in range(HC // 16)]
        tail_idx = jnp.minimum(iota, H - 1)
        dnums = lax.GatherDimensionNumbers(
            offset_dims=(), collapsed_slice_dims=(0,), start_index_map=(0,))

        def take16(v, idx):
            return lax.gather(v, idx[:, None], dnums, (1,),
                              mode=lax.GatherScatterMode.PROMISE_IN_BOUNDS)

        def zbody(r, _):
            for j in range(W_ACC // 16):
                cp_v[r, pl.ds(j * 16, 16)] = zeros16
            return 0
        lax.fori_loop(0, CPR, zbody, 0)
        for t in range(TMAX):
            cid = t * NS + s

            @pl.when(cid < NCPT)
            def _():
                pltpu.sync_copy(cp_v, acc_sh.at[pl.ds(cid * CPR, CPR)])
        plsc.subcore_barrier()

        wid = s * NC + c
        base = wid * EW

        def chunk(ci, _):
            off = base + ci * K
            d1 = pltpu.async_copy(src_hbm.at[pl.ds(off, K)], src_v, sem_a)
            d2 = pltpu.async_copy(dst_hbm.at[pl.ds(off, K)], dst_v, sem_a)
            d3 = pltpu.async_copy(ae_hbm.at[pl.ds(off * H, K * H)],
                                  ae_v.at[pl.ds(0, K * H)], sem_a)
            d1.wait(); d2.wait(); d3.wait()
            g1 = pltpu.async_copy(t_hbm.at[src_v], t_v, sem_b)
            g2 = pltpu.async_copy(ad_hbm.at[dst_v], ad_v, sem_b)
            g1.wait(); g2.wait()

            def mbody(k, _):
                a = (t_v[k, pl.ds(HC, 16)] + ad_v[k, pl.ds(0, 16)]
                     + ae_v[pl.ds(k * H, 16)])
                a = jnp.maximum(a, 0.2 * a)
                w = jnp.exp(a)
                for j in range(HC // 16):
                    x = t_v[k, pl.ds(j * 16, 16)]
                    msg_v[k, pl.ds(j * 16, 16)] = x * take16(w, hol[j])
                wt = take16(w, tail_idx)
                msg_v[k, pl.ds(HC, 16)] = jnp.where(iota < H, wt, 0.0)
                return 0
            lax.fori_loop(0, K, mbody, 0)

            pltpu.sync_copy(msg_v, acc_sh.at[dst_v], add=True)
            return 0
        lax.fori_loop(0, NCH, chunk, 0)

        plsc.subcore_barrier()
        for t in range(TMAX):
            cid = t * NS + s

            @pl.when(cid < NCPT)
            def _():
                r0 = cid * CPR
                pltpu.sync_copy(acc_sh.at[pl.ds(r0, CPR)], cp_v)
                pltpu.sync_copy(cp_v, out_hbm.at[c, pl.ds(r0, CPR)])

    return body


def _stage_a_kernel(x_ref, wc_ref, wd_ref, t_ref, ad_ref):
    x = x_ref[...]
    t_ref[...] = jnp.dot(x, wc_ref[...], preferred_element_type=jnp.float32)
    ad_ref[...] = jnp.dot(x, wd_ref[...], preferred_element_type=jnp.float32)


def _stage_b_kernel(ea_ref, w1_ref, w2_ref, o1_ref, o2_ref):
    ea = ea_ref[...]
    o1_ref[...] = jnp.dot(ea, w1_ref[...], preferred_element_type=jnp.float32)
    o2_ref[...] = jnp.dot(ea, w2_ref[...], preferred_element_type=jnp.float32)


def _stage_c_kernel(HC, H, acc_ref, b_ref, p_ref, wc_ref, wd_ref, t_ref, ad_ref):
    a = acc_ref[...]
    st = a[0] + a[1]
    msg = st[:, :HC]
    den = st[:, HC:HC + H]
    recip = 1.0 / (den + 1e-16)
    h = msg * jnp.dot(recip, p_ref[...], preferred_element_type=jnp.float32)
    h = h + b_ref[...]
    h = jnp.where(h > 0, h, jnp.exp(jnp.minimum(h, 0.0)) - 1.0)
    t_ref[...] = jnp.dot(h, wc_ref[...], preferred_element_type=jnp.float32)
    ad_ref[...] = jnp.dot(h, wd_ref[...], preferred_element_type=jnp.float32)


def _stage_d_kernel(HC, G, acc_ref, b_ref, batch_ref, o_ref):
    a = acc_ref[...]
    st = a[0] + a[1]
    h = st[:, :HC] / (st[:, HC:HC + 1] + 1e-16) + b_ref[...]
    n = h.shape[0]
    oh = (batch_ref[...] == lax.broadcasted_iota(jnp.int32, (1, G), 1))
    oh = oh.astype(jnp.float32)
    sums = lax.dot_general(oh, h, (((0,), (0,)), ((), ())),
                           preferred_element_type=jnp.float32)
    cnt = lax.dot_general(oh, jnp.ones((n, 1), jnp.float32),
                          (((0,), (0,)), ((), ())),
                          preferred_element_type=jnp.float32)
    o_ref[...] = sums / jnp.clip(cnt, 1.0, None)


def _full_spec(shape):
    return pl.BlockSpec(shape, lambda: tuple(0 for _ in shape))


def _dense_call(body, ins, out_shapes):
    return pl.pallas_call(
        body,
        in_specs=[_full_spec(a.shape) for a in ins],
        out_specs=tuple(_full_spec(s.shape) for s in out_shapes),
        out_shape=tuple(out_shapes),
    )(*ins)


def _block_diag_att(att):
    h, c = att.shape
    return (att[:, :, None] * jnp.eye(h, dtype=att.dtype)[:, None, :]).reshape(h * c, h)


def kernel(x, edge_index, batch, edge_attr,
           W1, att_src1, att_dst1, We1, att_e1, b1,
           W2, att_src2, att_dst2, We2, att_e2, b2):
    N, F = x.shape
    E = edge_index.shape[1]
    H1, C1 = att_src1.shape[1], att_src1.shape[2]
    H2, C2 = att_src2.shape[1], att_src2.shape[2]
    HC1, HC2 = H1 * C1, H2 * C2
    G = 64
    src = edge_index[0]
    dst = edge_index[1]

    f32 = jnp.float32
    Ws1 = W1 @ _block_diag_att(att_src1[0])
    Wd1 = W1 @ _block_diag_att(att_dst1[0])
    Wcat1 = jnp.concatenate([W1, Ws1, jnp.zeros((F, 16 - H1), f32)], axis=1)
    Wdcat1 = jnp.concatenate([Wd1, jnp.zeros((F, 16 - H1), f32)], axis=1)
    Weff1 = jnp.einsum("dhc,hc->dh", We1.reshape(-1, H1, C1), att_e1[0])
    Ws2 = W2 @ _block_diag_att(att_src2[0])
    Wd2 = W2 @ _block_diag_att(att_dst2[0])
    Wcat2 = jnp.concatenate([W2, Ws2, jnp.zeros((HC1, 16 - H2), f32)], axis=1)
    Wdcat2 = jnp.concatenate([Wd2, jnp.zeros((HC1, 16 - H2), f32)], axis=1)
    Weff2 = jnp.einsum("dhc,hc->dh", We2.reshape(-1, H2, C2), att_e2[0])
    P1 = jnp.kron(jnp.eye(H1, dtype=f32), jnp.ones((1, C1), f32))

    T1, AD1 = _dense_call(
        _stage_a_kernel, [x, Wcat1, Wdcat1],
        [jax.ShapeDtypeStruct((N, HC1 + 16), f32),
         jax.ShapeDtypeStruct((N, 16), f32)])

    EB = 4000
    ae1, ae2 = pl.pallas_call(
        _stage_b_kernel,
        grid=(E // EB,),
        in_specs=[pl.BlockSpec((EB, edge_attr.shape[1]), lambda i: (i, 0)),
                  pl.BlockSpec(Weff1.shape, lambda i: (0, 0)),
                  pl.BlockSpec(Weff2.shape, lambda i: (0, 0))],
        out_specs=(pl.BlockSpec((EB, H1), lambda i: (i, 0)),
                   pl.BlockSpec((EB, H2), lambda i: (i, 0))),
        out_shape=(jax.ShapeDtypeStruct((E, H1), f32),
                   jax.ShapeDtypeStruct((E, H2), f32)),
    )(edge_attr, Weff1, Weff2)

    acc1 = _sc_edge_aggregate(H1, C1, N, E)(T1, AD1, ae1.reshape(-1), src, dst)

    T2, AD2 = _dense_call(
        functools.partial(_stage_c_kernel, HC1, H1),
        [acc1, b1.reshape(1, -1), P1, Wcat2, Wdcat2],
        [jax.ShapeDtypeStruct((N, HC2 + 16), f32),
         jax.ShapeDtypeStruct((N, 16), f32)])

    acc2 = _sc_edge_aggregate(H2, C2, N, E)(T2, AD2, ae2.reshape(-1), src, dst)

    (out,) = _dense_call(
        functools.partial(_stage_d_kernel, HC2, G),
        [acc2, b2.reshape(1, -1), batch.reshape(-1, 1)],
        [jax.ShapeDtypeStruct((G, HC2), f32)])
    return out


# speedup vs baseline: 46.4362x; 1.2810x over previous
import functools

import jax
import jax.numpy as jnp
from jax import lax
from jax.experimental import pallas as pl
from jax.experimental.pallas import tpu as pltpu
from jax.experimental.pallas import tpu_sc as plsc

NC = 2
NS = 16
NW = NC * NS


def _sc_edge_aggregate(H, C, N, E):
    HC = H * C
    W_ACC = HC + 16
    K = 80 if W_ACC <= 96 else 40
    EW = E // NW
    NCH = EW // K
    assert EW * NW == E and NCH * K == EW and NCH >= 5
    CPR = K
    NCPT = N // CPR
    TMAX = -(-NCPT // NS)
    assert NCPT * CPR == N
    mesh = plsc.VectorSubcoreMesh(core_axis_name="c", subcore_axis_name="s",
                                  num_cores=NC, num_subcores=NS)

    @functools.partial(
        pl.kernel,
        out_type=jax.ShapeDtypeStruct((NC, N, W_ACC), jnp.float32),
        mesh=mesh,
        compiler_params=pltpu.CompilerParams(use_tc_tiling_on_sc=False),
        scratch_types=[
            pltpu.VMEM((K,), jnp.int32),
            pltpu.VMEM((K,), jnp.int32),
            pltpu.VMEM((K,), jnp.int32),
            pltpu.VMEM((K,), jnp.int32),
            pltpu.VMEM((K,), jnp.int32),
            pltpu.VMEM((K,), jnp.int32),
            pltpu.VMEM((K * H + 16,), jnp.float32),
            pltpu.VMEM((K * H + 16,), jnp.float32),
            pltpu.VMEM((K, W_ACC), jnp.float32),
            pltpu.VMEM((K, W_ACC), jnp.float32),
            pltpu.VMEM((K, 16), jnp.float32),
            pltpu.VMEM((K, 16), jnp.float32),
            pltpu.VMEM((K, W_ACC), jnp.float32),
            pltpu.VMEM((K, W_ACC), jnp.float32),
            pltpu.VMEM((CPR, W_ACC), jnp.float32),
            pltpu.VMEM_SHARED((N, W_ACC), jnp.float32),
            pltpu.SemaphoreType.DMA,
            pltpu.SemaphoreType.DMA,
            pltpu.SemaphoreType.DMA,
            pltpu.SemaphoreType.DMA,
        ],
    )
    def body(t_hbm, ad_hbm, ae_hbm, src_hbm, dst_hbm, out_hbm,
             src_v0, src_v1, dst_v0, dst_v1, dsc_v0, dsc_v1,
             ae_v0, ae_v1, t_v0, t_v1, ad_v0, ad_v1, msg_v0, msg_v1,
             cp_v, acc_sh, sem_l, sem_g, sem_s0, sem_s1):
        src_v = [src_v0, src_v1]
        dst_v = [dst_v0, dst_v1]
        dsc_v = [dsc_v0, dsc_v1]
        ae_v = [ae_v0, ae_v1]
        t_v = [t_v0, t_v1]
        ad_v = [ad_v0, ad_v1]
        msg_v = [msg_v0, msg_v1]
        sem_s = [sem_s0, sem_s1]
        c = lax.axis_index("c")
        s = lax.axis_index("s")
        iota = lax.iota(jnp.int32, 16)
        zeros16 = jnp.zeros((16,), jnp.float32)
        log2c = C.bit_length() - 1
        hol = [(j * 16 + iota) >> log2c for j in range(HC // 16)]
        tail_idx = jnp.minimum(iota, H - 1)
        dnums = lax.GatherDimensionNumbers(
            offset_dims=(), collapsed_slice_dims=(0,), start_index_map=(0,))

        def take16(v, idx):
            return lax.gather(v, idx[:, None], dnums, (1,),
                              mode=lax.GatherScatterMode.PROMISE_IN_BOUNDS)

        def zbody(r, _):
            for j in range(W_ACC // 16):
                cp_v[r, pl.ds(j * 16, 16)] = zeros16
            return 0
        lax.fori_loop(0, CPR, zbody, 0)
        for t in range(TMAX):
            cid = t * NS + s

            @pl.when(cid < NCPT)
            def _():
                pltpu.sync_copy(cp_v, acc_sh.at[pl.ds(cid * CPR, CPR)])
        plsc.subcore_barrier()

        wid = s * NC + c
        base = wid * EW

        def issue_l(ci, b):
            off = base + ci * K
            pltpu.async_copy(src_hbm.at[pl.ds(off, K)], src_v[b], sem_l)
            pltpu.async_copy(dst_hbm.at[pl.ds(off, K)], dst_v[b], sem_l)
            pltpu.async_copy(ae_hbm.at[pl.ds(off * H, K * H)],
                             ae_v[b].at[pl.ds(0, K * H)], sem_l)

        def wait_l(ci, b):
            off = base + ci * K
            pltpu.make_async_copy(src_hbm.at[pl.ds(off, K)], src_v[b], sem_l).wait()
            pltpu.make_async_copy(dst_hbm.at[pl.ds(off, K)], dst_v[b], sem_l).wait()
            pltpu.make_async_copy(ae_hbm.at[pl.ds(off * H, K * H)],
                                  ae_v[b].at[pl.ds(0, K * H)], sem_l).wait()

        def issue_g(b):
            pltpu.async_copy(t_hbm.at[src_v[b]], t_v[b], sem_g)
            pltpu.async_copy(ad_hbm.at[dst_v[b]], ad_v[b], sem_g)

        def wait_g(b):
            pltpu.make_async_copy(t_hbm.at[src_v[b]], t_v[b], sem_g).wait()
            pltpu.make_async_copy(ad_hbm.at[dst_v[b]], ad_v[b], sem_g).wait()

        def wait_s(b):
            pltpu.make_async_copy(msg_v[b], acc_sh.at[dsc_v[b]], sem_s[b]).wait()

        def process(ci, b, first, last):
            nb = 1 - b
            wait_g(b)
            if not last:
                wait_l(ci + 1, nb)
                issue_g(nb)

            if not first:
                wait_s(b)
            starts = list(range(0, K - 15, 16))
            if K % 16:
                starts.append(K - 16)
            for j0 in starts:
                dsc_v[b][pl.ds(j0, 16)] = dst_v[b][pl.ds(j0, 16)]

            def mbody(k, _):
                tb, ab, eb, mb = t_v[b], ad_v[b], ae_v[b], msg_v[b]
                a = (tb[k, pl.ds(HC, 16)] + ab[k, pl.ds(0, 16)]
                     + eb[pl.ds(k * H, 16)])
                a = jnp.maximum(a, 0.2 * a)
                w = jnp.exp(a)
                for j in range(HC // 16):
                    mb[k, pl.ds(j * 16, 16)] = tb[k, pl.ds(j * 16, 16)] * take16(w, hol[j])
                wt = take16(w, tail_idx)
                mb[k, pl.ds(HC, 16)] = jnp.where(iota < H, wt, 0.0)
                return 0
            lax.fori_loop(0, K, mbody, 0)

            pltpu.async_copy(msg_v[b], acc_sh.at[dsc_v[b]], sem_s[b], add=True)
            if not last:
                if isinstance(ci, int):
                    if ci + 2 < NCH:
                        issue_l(ci + 2, b)
                else:
                    @pl.when(ci + 2 < NCH)
                    def _():
                        issue_l(ci + 2, b)

        issue_l(0, 0)
        wait_l(0, 0)
        issue_g(0)
        issue_l(1, 1)

        def pair(i, _):
            process(2 * i, 0, first=False, last=False)
            process(2 * i + 1, 1, first=False, last=False)
            return 0

        process(0, 0, first=True, last=False)
        process(1, 1, first=True, last=False)
        if NCH % 2 == 1:
            lax.fori_loop(1, (NCH - 1) // 2, pair, 0)
            process(NCH - 1, 0, first=False, last=True)
            wait_s(1)
            wait_s(0)
        else:
            lax.fori_loop(1, NCH // 2 - 1, pair, 0)
            process(NCH - 2, 0, first=False, last=False)
            process(NCH - 1, 1, first=False, last=True)
            wait_s(0)
            wait_s(1)

        plsc.subcore_barrier()
        for t in range(TMAX):
            cid = t * NS + s

            @pl.when(cid < NCPT)
            def _():
                r0 = cid * CPR
                pltpu.sync_copy(acc_sh.at[pl.ds(r0, CPR)], cp_v)
                pltpu.sync_copy(cp_v, out_hbm.at[c, pl.ds(r0, CPR)])

    return body


def _stage_a_kernel(x_ref, wc_ref, wd_ref, t_ref, ad_ref):
    x = x_ref[...]
    t_ref[...] = jnp.dot(x, wc_ref[...], preferred_element_type=jnp.float32)
    ad_ref[...] = jnp.dot(x, wd_ref[...], preferred_element_type=jnp.float32)


def _stage_b_kernel(ea_ref, w1_ref, w2_ref, o1_ref, o2_ref):
    ea = ea_ref[...]
    o1_ref[...] = jnp.dot(ea, w1_ref[...], preferred_element_type=jnp.float32)
    o2_ref[...] = jnp.dot(ea, w2_ref[...], preferred_element_type=jnp.float32)


def _stage_c_kernel(HC, H, acc_ref, b_ref, p_ref, wc_ref, wd_ref, t_ref, ad_ref):
    a = acc_ref[...]
    st = a[0] + a[1]
    msg = st[:, :HC]
    den = st[:, HC:HC + H]
    recip = 1.0 / (den + 1e-16)
    h = msg * jnp.dot(recip, p_ref[...], preferred_element_type=jnp.float32)
    h = h + b_ref[...]
    h = jnp.where(h > 0, h, jnp.exp(jnp.minimum(h, 0.0)) - 1.0)
    t_ref[...] = jnp.dot(h, wc_ref[...], preferred_element_type=jnp.float32)
    ad_ref[...] = jnp.dot(h, wd_ref[...], preferred_element_type=jnp.float32)


def _stage_d_kernel(HC, G, acc_ref, b_ref, batch_ref, o_ref):
    a = acc_ref[...]
    st = a[0] + a[1]
    h = st[:, :HC] / (st[:, HC:HC + 1] + 1e-16) + b_ref[...]
    n = h.shape[0]
    oh = (batch_ref[...] == lax.broadcasted_iota(jnp.int32, (1, G), 1))
    oh = oh.astype(jnp.float32)
    sums = lax.dot_general(oh, h, (((0,), (0,)), ((), ())),
                           preferred_element_type=jnp.float32)
    cnt = lax.dot_general(oh, jnp.ones((n, 1), jnp.float32),
                          (((0,), (0,)), ((), ())),
                          preferred_element_type=jnp.float32)
    o_ref[...] = sums / jnp.clip(cnt, 1.0, None)


def _full_spec(shape):
    return pl.BlockSpec(shape, lambda: tuple(0 for _ in shape))


def _dense_call(body, ins, out_shapes):
    return pl.pallas_call(
        body,
        in_specs=[_full_spec(a.shape) for a in ins],
        out_specs=tuple(_full_spec(s.shape) for s in out_shapes),
        out_shape=tuple(out_shapes),
    )(*ins)


def _block_diag_att(att):
    h, c = att.shape
    return (att[:, :, None] * jnp.eye(h, dtype=att.dtype)[:, None, :]).reshape(h * c, h)


def kernel(x, edge_index, batch, edge_attr,
           W1, att_src1, att_dst1, We1, att_e1, b1,
           W2, att_src2, att_dst2, We2, att_e2, b2):
    N, F = x.shape
    E = edge_index.shape[1]
    H1, C1 = att_src1.shape[1], att_src1.shape[2]
    H2, C2 = att_src2.shape[1], att_src2.shape[2]
    HC1, HC2 = H1 * C1, H2 * C2
    G = 64
    src = edge_index[0]
    dst = edge_index[1]

    f32 = jnp.float32
    Ws1 = W1 @ _block_diag_att(att_src1[0])
    Wd1 = W1 @ _block_diag_att(att_dst1[0])
    Wcat1 = jnp.concatenate([W1, Ws1, jnp.zeros((F, 16 - H1), f32)], axis=1)
    Wdcat1 = jnp.concatenate([Wd1, jnp.zeros((F, 16 - H1), f32)], axis=1)
    Weff1 = jnp.einsum("dhc,hc->dh", We1.reshape(-1, H1, C1), att_e1[0])
    Ws2 = W2 @ _block_diag_att(att_src2[0])
    Wd2 = W2 @ _block_diag_att(att_dst2[0])
    Wcat2 = jnp.concatenate([W2, Ws2, jnp.zeros((HC1, 16 - H2), f32)], axis=1)
    Wdcat2 = jnp.concatenate([Wd2, jnp.zeros((HC1, 16 - H2), f32)], axis=1)
    Weff2 = jnp.einsum("dhc,hc->dh", We2.reshape(-1, H2, C2), att_e2[0])
    P1 = jnp.kron(jnp.eye(H1, dtype=f32), jnp.ones((1, C1), f32))

    T1, AD1 = _dense_call(
        _stage_a_kernel, [x, Wcat1, Wdcat1],
        [jax.ShapeDtypeStruct((N, HC1 + 16), f32),
         jax.ShapeDtypeStruct((N, 16), f32)])

    EB = 4000
    ae1, ae2 = pl.pallas_call(
        _stage_b_kernel,
        grid=(E // EB,),
        in_specs=[pl.BlockSpec((EB, edge_attr.shape[1]), lambda i: (i, 0)),
                  pl.BlockSpec(Weff1.shape, lambda i: (0, 0)),
                  pl.BlockSpec(Weff2.shape, lambda i: (0, 0))],
        out_specs=(pl.BlockSpec((EB, H1), lambda i: (i, 0)),
                   pl.BlockSpec((EB, H2), lambda i: (i, 0))),
        out_shape=(jax.ShapeDtypeStruct((E, H1), f32),
                   jax.ShapeDtypeStruct((E, H2), f32)),
    )(edge_attr, Weff1, Weff2)

    acc1 = _sc_edge_aggregate(H1, C1, N, E)(T1, AD1, ae1.reshape(-1), src, dst)

    T2, AD2 = _dense_call(
        functools.partial(_stage_c_kernel, HC1, H1),
        [acc1, b1.reshape(1, -1), P1, Wcat2, Wdcat2],
        [jax.ShapeDtypeStruct((N, HC2 + 16), f32),
         jax.ShapeDtypeStruct((N, 16), f32)])

    acc2 = _sc_edge_aggregate(H2, C2, N, E)(T2, AD2, ae2.reshape(-1), src, dst)

    (out,) = _dense_call(
        functools.partial(_stage_d_kernel, HC2, G),
        [acc2, b2.reshape(1, -1), batch.reshape(-1, 1)],
        [jax.ShapeDtypeStruct((G, HC2), f32)])
    return out
